# speedup vs baseline: 3.5224x; 3.5224x over previous
import functools

import jax
import jax.numpy as jnp
from jax import lax
from jax.experimental import pallas as pl
from jax.experimental.pallas import tpu as pltpu
from jax.experimental.pallas import tpu_sc as plsc

N_NODES = 10000
N_EDGES = 160000
H_SIZE = 256
HALF = 128

NPAD = 10240
ROWS_PER_SUB = NPAD // 16
EDGES_PER_SUB = N_EDGES // 16
K = 80
NCHUNK = EDGES_PER_SUB // K


def _sc_body(h0_hbm, h1_hbm, src_hbm, dst_hbm, zeros_hbm, out_hbm,
             acc_sh, src_v, dst_v, rows_v, sem):
    c = lax.axis_index("c")
    s = lax.axis_index("s")
    row0 = s * ROWS_PER_SUB

    pltpu.sync_copy(zeros_hbm, acc_sh.at[pl.ds(row0, ROWS_PER_SUB)])
    plsc.subcore_barrier()

    ebase = s * EDGES_PER_SUB

    def chunk(i, carry):
        off = ebase + i * K
        pltpu.sync_copy(src_hbm.at[pl.ds(off, K)], src_v)
        pltpu.sync_copy(dst_hbm.at[pl.ds(off, K)], dst_v)

        @pl.when(c == 0)
        def _():
            pltpu.async_copy(h0_hbm.at[src_v], rows_v, sem).wait()

        @pl.when(c == 1)
        def _():
            pltpu.async_copy(h1_hbm.at[src_v], rows_v, sem).wait()

        pltpu.sync_copy(rows_v, acc_sh.at[dst_v], add=True)
        return carry

    lax.fori_loop(0, NCHUNK, chunk, 0)
    plsc.subcore_barrier()

    out0 = c * NPAD + row0
    pltpu.sync_copy(acc_sh.at[pl.ds(row0, ROWS_PER_SUB)],
                    out_hbm.at[pl.ds(out0, ROWS_PER_SUB)])


_sc_segment_sum = functools.partial(
    pl.kernel,
    out_type=jax.ShapeDtypeStruct((2 * NPAD, HALF), jnp.float32),
    mesh=plsc.VectorSubcoreMesh(core_axis_name="c", subcore_axis_name="s"),
    scratch_types=[
        pltpu.VMEM_SHARED((NPAD, HALF), jnp.float32),
        pltpu.VMEM((K,), jnp.int32),
        pltpu.VMEM((K,), jnp.int32),
        pltpu.VMEM((K, HALF), jnp.float32),
        pltpu.SemaphoreType.DMA,
    ],
)(_sc_body)


ROW_BLK = 1000


def _tc_body(x_ref, ht0_ref, ht1_ref, w_ref, u_ref, b_ref, out_ref):
    ht = jnp.concatenate([ht0_ref[...], ht1_ref[...]], axis=1)
    ruo = (jnp.dot(x_ref[...], w_ref[...], preferred_element_type=jnp.float32)
           + jnp.dot(ht, u_ref[...], preferred_element_type=jnp.float32)
           + b_ref[...])
    u = jax.nn.sigmoid(ruo[:, :H_SIZE])
    o = jnp.tanh(ruo[:, H_SIZE:])
    out_ref[...] = o * u + (1.0 - u) * ht


_tc_dense = pl.pallas_call(
    _tc_body,
    out_shape=jax.ShapeDtypeStruct((N_NODES, H_SIZE), jnp.float32),
    grid=(N_NODES // ROW_BLK,),
    in_specs=[
        pl.BlockSpec((ROW_BLK, H_SIZE), lambda i: (i, 0)),
        pl.BlockSpec((ROW_BLK, HALF), lambda i: (i, 0)),
        pl.BlockSpec((ROW_BLK, HALF), lambda i: (i, 0)),
        pl.BlockSpec((H_SIZE, 2 * H_SIZE), lambda i: (0, 0)),
        pl.BlockSpec((H_SIZE, 2 * H_SIZE), lambda i: (0, 0)),
        pl.BlockSpec((1, 2 * H_SIZE), lambda i: (0, 0)),
    ],
    out_specs=pl.BlockSpec((ROW_BLK, H_SIZE), lambda i: (i, 0)),
)


def kernel(x, h, edge_index, W_ruo, U_ruo, b_ruo):
    src = edge_index[0].astype(jnp.int32)
    dst = edge_index[1].astype(jnp.int32)
    hr = h.reshape(N_NODES, 2, HALF).transpose(1, 0, 2)
    zeros = jnp.zeros((ROWS_PER_SUB, HALF), jnp.float32)

    ht_flat = _sc_segment_sum(hr[0], hr[1], src, dst, zeros)
    ht0 = ht_flat[:N_NODES]
    ht1 = ht_flat[NPAD:NPAD + N_NODES]

    W2 = W_ruo[:, H_SIZE:]
    U2 = U_ruo[:, H_SIZE:]
    b2 = b_ruo[:, H_SIZE:]
    return _tc_dense(x, ht0, ht1, W2, U2, b2)
